# baseline (speedup 1.0000x reference)
import functools

import jax
import jax.numpy as jnp
from jax import lax
from jax.experimental import pallas as pl
from jax.experimental.pallas import tpu as pltpu
from jax.experimental.pallas import tpu_sc as plsc

_N = 20000
_MAX_DET = 100
_IOU_THRESH = 0.5
_SCORE_THRESH = 0.05

_NTILES = 16
_PERTILE = 1280
_NPAD = _NTILES * _PERTILE
_VPT = _PERTILE // 16
_NBINS = 128
_C0 = 192
_SLAB = 32
_NV0 = 24
_FULLBIN = int(_SCORE_THRESH * _NBINS) + 1
_STG = _PERTILE + 16
_BIG = 2**30

_mesh = plsc.VectorSubcoreMesh(core_axis_name="c", subcore_axis_name="s",
                               num_cores=1)


@functools.partial(
    pl.kernel,
    out_type=jax.ShapeDtypeStruct((_MAX_DET * 16,), jnp.float32),
    mesh=_mesh,
    compiler_params=pltpu.CompilerParams(needs_layout_passes=False),
    scratch_types=dict(
        x1_v=pltpu.VMEM((_PERTILE,), jnp.float32),
        y1_v=pltpu.VMEM((_PERTILE,), jnp.float32),
        x2_v=pltpu.VMEM((_PERTILE,), jnp.float32),
        y2_v=pltpu.VMEM((_PERTILE,), jnp.float32),
        s_v=pltpu.VMEM((_PERTILE,), jnp.float32),
        hist_v=pltpu.VMEM((_NBINS,), jnp.float32),
        histall_v=pltpu.VMEM((_NTILES * _NBINS,), jnp.float32),
        recall_v=pltpu.VMEM((_NTILES * 16,), jnp.float32),
        stg_x1=pltpu.VMEM((_STG,), jnp.float32),
        stg_y1=pltpu.VMEM((_STG,), jnp.float32),
        stg_x2=pltpu.VMEM((_STG,), jnp.float32),
        stg_y2=pltpu.VMEM((_STG,), jnp.float32),
        stg_s=pltpu.VMEM((_STG,), jnp.float32),
        cx1_v=pltpu.VMEM((_NPAD,), jnp.float32),
        cy1_v=pltpu.VMEM((_NPAD,), jnp.float32),
        cx2_v=pltpu.VMEM((_NPAD,), jnp.float32),
        cy2_v=pltpu.VMEM((_NPAD,), jnp.float32),
        cs_v=pltpu.VMEM((_NPAD,), jnp.float32),
        out_v=pltpu.VMEM((_MAX_DET * 16,), jnp.float32),
        rec_v=pltpu.VMEM((16,), jnp.float32),
        flag_v=pltpu.VMEM((16,), jnp.float32),
        hist_sh=pltpu.VMEM_SHARED((_NTILES * _NBINS,), jnp.float32),
        rec_sh=pltpu.VMEM_SHARED((_NTILES * 16,), jnp.float32),
        flag_sh=pltpu.VMEM_SHARED((16,), jnp.float32),
        cx1_sh=pltpu.VMEM_SHARED((_NPAD,), jnp.float32),
        cy1_sh=pltpu.VMEM_SHARED((_NPAD,), jnp.float32),
        cx2_sh=pltpu.VMEM_SHARED((_NPAD,), jnp.float32),
        cy2_sh=pltpu.VMEM_SHARED((_NPAD,), jnp.float32),
        cs_sh=pltpu.VMEM_SHARED((_NPAD,), jnp.float32),
    ),
)
def _sc_nms(x1_h, y1_h, x2_h, y2_h, s_h, out_h, *,
            x1_v, y1_v, x2_v, y2_v, s_v, hist_v, histall_v, recall_v,
            stg_x1, stg_y1, stg_x2, stg_y2, stg_s,
            cx1_v, cy1_v, cx2_v, cy2_v, cs_v, out_v, rec_v, flag_v,
            hist_sh, rec_sh, flag_sh,
            cx1_sh, cy1_sh, cx2_sh, cy2_sh, cs_sh):
    tid = lax.axis_index("s")
    lane = lax.iota(jnp.int32, 16)
    lanef = lane.astype(jnp.float32)
    base = tid * _PERTILE

    pltpu.sync_copy(x1_h.at[pl.ds(base, _PERTILE)], x1_v)
    pltpu.sync_copy(y1_h.at[pl.ds(base, _PERTILE)], y1_v)
    pltpu.sync_copy(x2_h.at[pl.ds(base, _PERTILE)], x2_v)
    pltpu.sync_copy(y2_h.at[pl.ds(base, _PERTILE)], y2_v)
    pltpu.sync_copy(s_h.at[pl.ds(base, _PERTILE)], s_v)

    zeros16 = jnp.zeros((16,), jnp.float32)
    ones16 = jnp.ones((16,), jnp.float32)

    def _zero(k, c):
        hist_v[pl.ds(k * 16, 16)] = zeros16
        return c
    lax.fori_loop(0, _NBINS // 16, _zero, 0)

    @plsc.parallel_loop(0, _VPT, unroll=4, carry=jnp.int32(0))
    def _hstep(j, c):
        o = j * 16
        s = s_v[pl.ds(o, 16)]
        sw = jnp.where(s > _SCORE_THRESH, s, -jnp.inf)
        s_v[pl.ds(o, 16)] = sw
        m = sw > _SCORE_THRESH
        b = jnp.clip((sw * float(_NBINS)).astype(jnp.int32), 0, _NBINS - 1)
        plsc.addupdate_scatter(hist_v, [b], ones16, mask=m)
        return c
    _ = _hstep

    pltpu.sync_copy(hist_v, hist_sh.at[pl.ds(tid * _NBINS, _NBINS)])
    plsc.subcore_barrier()
    pltpu.sync_copy(hist_sh, histall_v)

    def _rsum(r, accs):
        return tuple(accs[k] + histall_v[pl.ds(r * _NBINS + k * 16, 16)]
                     for k in range(_NBINS // 16))
    hs = lax.fori_loop(0, _NTILES, _rsum,
                       tuple(zeros16 for _ in range(_NBINS // 16)))

    total = jnp.float32(0.0)
    bstar = jnp.int32(-1)
    for k in range(_NBINS // 16 - 1, -1, -1):
        v = hs[k]
        suf = lax.rev(plsc.cumsum(lax.rev(v, (0,))), (0,)) + total
        mask = suf >= float(_C0)
        bins = jnp.int32(k * 16) + lane
        bstar = jnp.maximum(bstar, jnp.max(jnp.where(mask, bins, -1)))
        total = total + jnp.sum(v)
    tthr = bstar.astype(jnp.float32) * (1.0 / _NBINS)

    for q in range(_SLAB // 16 + 1):
        stg_s[pl.ds(q * 16, 16)] = jnp.full((16,), -jnp.inf)

    def _cstep(j, off):
        o = j * 16
        sw = s_v[pl.ds(o, 16)]
        m = sw > tthr
        plsc.store_compressed(stg_s.at[pl.ds(off, 16)], sw, mask=m)
        plsc.store_compressed(stg_x1.at[pl.ds(off, 16)],
                              x1_v[pl.ds(o, 16)], mask=m)
        plsc.store_compressed(stg_y1.at[pl.ds(off, 16)],
                              y1_v[pl.ds(o, 16)], mask=m)
        plsc.store_compressed(stg_x2.at[pl.ds(off, 16)],
                              x2_v[pl.ds(o, 16)], mask=m)
        plsc.store_compressed(stg_y2.at[pl.ds(off, 16)],
                              y2_v[pl.ds(o, 16)], mask=m)
        return off + jnp.max(plsc.all_reduce_population_count(m))
    kcnt = lax.fori_loop(0, _VPT, _cstep, jnp.int32(0))

    pltpu.sync_copy(stg_x1.at[pl.ds(0, _SLAB)],
                    cx1_sh.at[pl.ds(tid * _SLAB, _SLAB)])
    pltpu.sync_copy(stg_y1.at[pl.ds(0, _SLAB)],
                    cy1_sh.at[pl.ds(tid * _SLAB, _SLAB)])
    pltpu.sync_copy(stg_x2.at[pl.ds(0, _SLAB)],
                    cx2_sh.at[pl.ds(tid * _SLAB, _SLAB)])
    pltpu.sync_copy(stg_y2.at[pl.ds(0, _SLAB)],
                    cy2_sh.at[pl.ds(tid * _SLAB, _SLAB)])
    pltpu.sync_copy(stg_s.at[pl.ds(0, _SLAB)],
                    cs_sh.at[pl.ds(tid * _SLAB, _SLAB)])
    rec_v[pl.ds(0, 16)] = jnp.where(lane == 0, kcnt.astype(jnp.float32), 0.0)
    pltpu.sync_copy(rec_v, rec_sh.at[pl.ds(tid * 16, 16)])
    plsc.subcore_barrier()

    def _nms_loop(nv, unroll):
        def _iter(i, carry):
            wx1, wy1, wx2, wy2, srv = carry
            warea = (wx2 - wx1) * (wy2 - wy1)

            @plsc.parallel_loop(0, nv, unroll=unroll,
                                carry=(jnp.full((16,), -jnp.inf),
                                       jnp.zeros((16,), jnp.int32)))
            def _scan(j, st):
                best, bidx = st
                o = j * 16
                x1 = cx1_v[pl.ds(o, 16)]
                y1 = cy1_v[pl.ds(o, 16)]
                x2 = cx2_v[pl.ds(o, 16)]
                y2 = cy2_v[pl.ds(o, 16)]
                s = cs_v[pl.ds(o, 16)]
                inter = (jnp.maximum(jnp.minimum(wx2, x2) -
                                     jnp.maximum(wx1, x1), 0.0) *
                         jnp.maximum(jnp.minimum(wy2, y2) -
                                     jnp.maximum(wy1, y1), 0.0))
                union = warea + (x2 - x1) * (y2 - y1) - inter
                iou = inter / jnp.maximum(union, 1e-9)
                snew = jnp.where(iou > _IOU_THRESH, -jnp.inf, s)
                cs_v[pl.ds(o, 16)] = snew
                jv = jnp.full((16,), j)
                upd = jnp.logical_or(
                    snew > best,
                    jnp.logical_and(snew == best, jv < bidx))
                best = jnp.where(upd, snew, best)
                bidx = jnp.where(upd, jv, bidx)
                return best, bidx

            best, bidx = _scan
            m = jnp.max(best)
            gidx = jnp.min(jnp.where(best == m, bidx * 16 + lane, _BIG))
            gidx = jnp.clip(gidx, 0, nv * 16 - 1)
            gv = jnp.full((16,), gidx)
            nwx1 = plsc.load_gather(cx1_v, [gv])
            nwy1 = plsc.load_gather(cy1_v, [gv])
            nwx2 = plsc.load_gather(cx2_v, [gv])
            nwy2 = plsc.load_gather(cy2_v, [gv])
            valid = m > -jnp.inf
            vb = jnp.full((16,), jnp.where(valid, 1.0, 0.0))
            vals = jnp.where(lane == 0, nwx1, 0.0)
            vals = jnp.where(lane == 1, nwy1, vals)
            vals = jnp.where(lane == 2, nwx2, vals)
            vals = jnp.where(lane == 3, nwy2, vals)
            vals = jnp.where(lane == 4, jnp.full((16,), m), vals)
            vals = jnp.where(vb > 0.5, vals, 0.0)
            out_v[pl.ds(i * 16, 16)] = vals
            return (nwx1, nwy1, nwx2, nwy2, srv + valid.astype(jnp.int32))

        init = (jnp.full((16,), -4000.0), jnp.full((16,), -4000.0),
                jnp.full((16,), -4000.0), jnp.full((16,), -4000.0),
                jnp.int32(0))
        return lax.fori_loop(0, _MAX_DET, _iter, init)[4]

    @pl.when(tid == 0)
    def _():
        nraw = _NTILES * _SLAB
        pltpu.sync_copy(cx1_sh.at[pl.ds(0, nraw)], cx1_v.at[pl.ds(0, nraw)])
        pltpu.sync_copy(cy1_sh.at[pl.ds(0, nraw)], cy1_v.at[pl.ds(0, nraw)])
        pltpu.sync_copy(cx2_sh.at[pl.ds(0, nraw)], cx2_v.at[pl.ds(0, nraw)])
        pltpu.sync_copy(cy2_sh.at[pl.ds(0, nraw)], cy2_v.at[pl.ds(0, nraw)])
        pltpu.sync_copy(cs_sh.at[pl.ds(0, nraw)], cs_v.at[pl.ds(0, nraw)])
        pltpu.sync_copy(rec_sh, recall_v)
        kvec = plsc.load_gather(recall_v, [lane * 16]).astype(jnp.int32)
        overflow = jnp.max(kvec) > _SLAB

        def _sq(j, off):
            o = j * 16
            sw = cs_v[pl.ds(o, 16)]
            m = sw > tthr
            plsc.store_compressed(cs_v.at[pl.ds(off, 16)], sw, mask=m)
            plsc.store_compressed(cx1_v.at[pl.ds(off, 16)],
                                  cx1_v[pl.ds(o, 16)], mask=m)
            plsc.store_compressed(cy1_v.at[pl.ds(off, 16)],
                                  cy1_v[pl.ds(o, 16)], mask=m)
            plsc.store_compressed(cx2_v.at[pl.ds(off, 16)],
                                  cx2_v[pl.ds(o, 16)], mask=m)
            plsc.store_compressed(cy2_v.at[pl.ds(off, 16)],
                                  cy2_v[pl.ds(o, 16)], mask=m)
            return off + jnp.max(plsc.all_reduce_population_count(m))
        ktot = lax.fori_loop(0, nraw // 16, _sq, jnp.int32(0))

        o0 = pl.multiple_of((ktot // 16) * 16, 16)
        tailv = cs_v[pl.ds(o0, 16)]
        cs_v[pl.ds(o0, 16)] = jnp.where(lane < ktot - o0, tailv, -jnp.inf)

        def _fill(j, c):
            cs_v[pl.ds(pl.multiple_of(o0 + 16 + j * 16, 16), 16)] = (
                jnp.full((16,), -jnp.inf))
            return c
        lax.fori_loop(0, jnp.maximum(_NV0 - o0 // 16 - 1, 0), _fill, 0)

        small = jnp.logical_and(jnp.logical_not(overflow), ktot <= 16 * 16)
        ok = jnp.logical_and(jnp.logical_not(overflow), ktot <= _NV0 * 16)
        srv = lax.cond(
            small, lambda: _nms_loop(16, 8),
            lambda: lax.cond(ok, lambda: _nms_loop(_NV0, 8),
                             lambda: jnp.int32(0)))
        need_full = jnp.logical_or(
            jnp.logical_not(ok),
            jnp.logical_and(srv < _MAX_DET, bstar >= _FULLBIN))
        flag_v[pl.ds(0, 16)] = jnp.full((16,), jnp.where(need_full, 1.0, 0.0))
        pltpu.sync_copy(flag_v, flag_sh)

    plsc.subcore_barrier()
    pltpu.sync_copy(flag_sh, flag_v)
    fl = jnp.max(flag_v[pl.ds(0, 16)])

    @pl.when(fl > 0.5)
    def _():
        pltpu.sync_copy(x1_v, cx1_sh.at[pl.ds(base, _PERTILE)])
        pltpu.sync_copy(y1_v, cy1_sh.at[pl.ds(base, _PERTILE)])
        pltpu.sync_copy(x2_v, cx2_sh.at[pl.ds(base, _PERTILE)])
        pltpu.sync_copy(y2_v, cy2_sh.at[pl.ds(base, _PERTILE)])
        pltpu.sync_copy(s_v, cs_sh.at[pl.ds(base, _PERTILE)])
        plsc.subcore_barrier()

        @pl.when(tid == 0)
        def _():
            pltpu.sync_copy(cx1_sh, cx1_v)
            pltpu.sync_copy(cy1_sh, cy1_v)
            pltpu.sync_copy(cx2_sh, cx2_v)
            pltpu.sync_copy(cy2_sh, cy2_v)
            pltpu.sync_copy(cs_sh, cs_v)
            _nms_loop(_NPAD // 16, 1)

    @pl.when(tid == 0)
    def _():
        pltpu.sync_copy(out_v, out_h)


@jax.jit
def kernel(boxes, scores):
    x1 = jnp.zeros((_NPAD,), jnp.float32).at[:_N].set(boxes[:, 0])
    y1 = jnp.zeros((_NPAD,), jnp.float32).at[:_N].set(boxes[:, 1])
    x2 = jnp.zeros((_NPAD,), jnp.float32).at[:_N].set(boxes[:, 2])
    y2 = jnp.zeros((_NPAD,), jnp.float32).at[:_N].set(boxes[:, 3])
    s = jnp.zeros((_NPAD,), jnp.float32).at[:_N].set(scores)
    out = _sc_nms(x1, y1, x2, y2, s)
    return out.reshape(_MAX_DET, 16)[:, :5]


# speedup vs baseline: 1.0134x; 1.0134x over previous
import functools

import jax
import jax.numpy as jnp
from jax import lax
from jax.experimental import pallas as pl
from jax.experimental.pallas import tpu as pltpu
from jax.experimental.pallas import tpu_sc as plsc

_N = 20000
_MAX_DET = 100
_IOU_THRESH = 0.5
_SCORE_THRESH = 0.05

_NTILES = 16
_PERTILE = 1280
_NPAD = _NTILES * _PERTILE
_VPT = _PERTILE // 16
_NBINS = 128
_C0 = 192
_SLAB = 32
_NV0 = 24
_FULLBIN = int(_SCORE_THRESH * _NBINS) + 1
_STG = _PERTILE + 16
_BIG = 2**30

_mesh = plsc.VectorSubcoreMesh(core_axis_name="c", subcore_axis_name="s",
                               num_cores=1)


@functools.partial(
    pl.kernel,
    out_type=jax.ShapeDtypeStruct((_MAX_DET * 16,), jnp.float32),
    mesh=_mesh,
    compiler_params=pltpu.CompilerParams(needs_layout_passes=False),
    scratch_types=dict(
        x1_v=pltpu.VMEM((_PERTILE,), jnp.float32),
        y1_v=pltpu.VMEM((_PERTILE,), jnp.float32),
        x2_v=pltpu.VMEM((_PERTILE,), jnp.float32),
        y2_v=pltpu.VMEM((_PERTILE,), jnp.float32),
        s_v=pltpu.VMEM((_PERTILE,), jnp.float32),
        hist_v=pltpu.VMEM((_NBINS,), jnp.float32),
        histall_v=pltpu.VMEM((_NTILES * _NBINS,), jnp.float32),
        recall_v=pltpu.VMEM((_NTILES * 16,), jnp.float32),
        stg_x1=pltpu.VMEM((_STG,), jnp.float32),
        stg_y1=pltpu.VMEM((_STG,), jnp.float32),
        stg_x2=pltpu.VMEM((_STG,), jnp.float32),
        stg_y2=pltpu.VMEM((_STG,), jnp.float32),
        stg_s=pltpu.VMEM((_STG,), jnp.float32),
        cx1_v=pltpu.VMEM((_NPAD,), jnp.float32),
        cy1_v=pltpu.VMEM((_NPAD,), jnp.float32),
        cx2_v=pltpu.VMEM((_NPAD,), jnp.float32),
        cy2_v=pltpu.VMEM((_NPAD,), jnp.float32),
        cs_v=pltpu.VMEM((_NPAD,), jnp.float32),
        out_v=pltpu.VMEM((_MAX_DET * 16,), jnp.float32),
        rec_v=pltpu.VMEM((16,), jnp.float32),
        flag_v=pltpu.VMEM((16,), jnp.float32),
        hist_sh=pltpu.VMEM_SHARED((_NTILES * _NBINS,), jnp.float32),
        rec_sh=pltpu.VMEM_SHARED((_NTILES * 16,), jnp.float32),
        flag_sh=pltpu.VMEM_SHARED((16,), jnp.float32),
        cx1_sh=pltpu.VMEM_SHARED((_NPAD,), jnp.float32),
        cy1_sh=pltpu.VMEM_SHARED((_NPAD,), jnp.float32),
        cx2_sh=pltpu.VMEM_SHARED((_NPAD,), jnp.float32),
        cy2_sh=pltpu.VMEM_SHARED((_NPAD,), jnp.float32),
        cs_sh=pltpu.VMEM_SHARED((_NPAD,), jnp.float32),
    ),
)
def _sc_nms(x1_h, y1_h, x2_h, y2_h, s_h, out_h, *,
            x1_v, y1_v, x2_v, y2_v, s_v, hist_v, histall_v, recall_v,
            stg_x1, stg_y1, stg_x2, stg_y2, stg_s,
            cx1_v, cy1_v, cx2_v, cy2_v, cs_v, out_v, rec_v, flag_v,
            hist_sh, rec_sh, flag_sh,
            cx1_sh, cy1_sh, cx2_sh, cy2_sh, cs_sh):
    tid = lax.axis_index("s")
    lane = lax.iota(jnp.int32, 16)
    lanef = lane.astype(jnp.float32)
    base = tid * _PERTILE

    pltpu.sync_copy(x1_h.at[pl.ds(base, _PERTILE)], x1_v)
    pltpu.sync_copy(y1_h.at[pl.ds(base, _PERTILE)], y1_v)
    pltpu.sync_copy(x2_h.at[pl.ds(base, _PERTILE)], x2_v)
    pltpu.sync_copy(y2_h.at[pl.ds(base, _PERTILE)], y2_v)
    pltpu.sync_copy(s_h.at[pl.ds(base, _PERTILE)], s_v)

    zeros16 = jnp.zeros((16,), jnp.float32)
    ones16 = jnp.ones((16,), jnp.float32)

    def _zero(k, c):
        hist_v[pl.ds(k * 16, 16)] = zeros16
        return c
    lax.fori_loop(0, _NBINS // 16, _zero, 0)

    @plsc.parallel_loop(0, _VPT, unroll=4, carry=jnp.int32(0))
    def _hstep(j, c):
        o = j * 16
        s = s_v[pl.ds(o, 16)]
        sw = jnp.where(s > _SCORE_THRESH, s, -jnp.inf)
        s_v[pl.ds(o, 16)] = sw
        m = sw > _SCORE_THRESH
        b = jnp.clip((sw * float(_NBINS)).astype(jnp.int32), 0, _NBINS - 1)
        plsc.addupdate_scatter(hist_v, [b], ones16, mask=m)
        return c
    _ = _hstep

    pltpu.sync_copy(hist_v, hist_sh.at[pl.ds(tid * _NBINS, _NBINS)])
    plsc.subcore_barrier()
    pltpu.sync_copy(hist_sh, histall_v)

    def _rsum(r, accs):
        return tuple(accs[k] + histall_v[pl.ds(r * _NBINS + k * 16, 16)]
                     for k in range(_NBINS // 16))
    hs = lax.fori_loop(0, _NTILES, _rsum,
                       tuple(zeros16 for _ in range(_NBINS // 16)))

    total = jnp.float32(0.0)
    bstar = jnp.int32(-1)
    for k in range(_NBINS // 16 - 1, -1, -1):
        v = hs[k]
        suf = lax.rev(plsc.cumsum(lax.rev(v, (0,))), (0,)) + total
        mask = suf >= float(_C0)
        bins = jnp.int32(k * 16) + lane
        bstar = jnp.maximum(bstar, jnp.max(jnp.where(mask, bins, -1)))
        total = total + jnp.sum(v)
    tthr = bstar.astype(jnp.float32) * (1.0 / _NBINS)

    for q in range(_SLAB // 16 + 1):
        stg_s[pl.ds(q * 16, 16)] = jnp.full((16,), -jnp.inf)

    def _cstep(j, off):
        o = j * 16
        sw = s_v[pl.ds(o, 16)]
        m = sw > tthr
        plsc.store_compressed(stg_s.at[pl.ds(off, 16)], sw, mask=m)
        plsc.store_compressed(stg_x1.at[pl.ds(off, 16)],
                              x1_v[pl.ds(o, 16)], mask=m)
        plsc.store_compressed(stg_y1.at[pl.ds(off, 16)],
                              y1_v[pl.ds(o, 16)], mask=m)
        plsc.store_compressed(stg_x2.at[pl.ds(off, 16)],
                              x2_v[pl.ds(o, 16)], mask=m)
        plsc.store_compressed(stg_y2.at[pl.ds(off, 16)],
                              y2_v[pl.ds(o, 16)], mask=m)
        return off + jnp.max(plsc.all_reduce_population_count(m))
    kcnt = lax.fori_loop(0, _VPT, _cstep, jnp.int32(0))

    pltpu.sync_copy(stg_x1.at[pl.ds(0, _SLAB)],
                    cx1_sh.at[pl.ds(tid * _SLAB, _SLAB)])
    pltpu.sync_copy(stg_y1.at[pl.ds(0, _SLAB)],
                    cy1_sh.at[pl.ds(tid * _SLAB, _SLAB)])
    pltpu.sync_copy(stg_x2.at[pl.ds(0, _SLAB)],
                    cx2_sh.at[pl.ds(tid * _SLAB, _SLAB)])
    pltpu.sync_copy(stg_y2.at[pl.ds(0, _SLAB)],
                    cy2_sh.at[pl.ds(tid * _SLAB, _SLAB)])
    pltpu.sync_copy(stg_s.at[pl.ds(0, _SLAB)],
                    cs_sh.at[pl.ds(tid * _SLAB, _SLAB)])
    rec_v[pl.ds(0, 16)] = jnp.where(lane == 0, kcnt.astype(jnp.float32), 0.0)
    pltpu.sync_copy(rec_v, rec_sh.at[pl.ds(tid * 16, 16)])
    plsc.subcore_barrier()

    def _nms_loop(nv, unroll):
        def _iter(i, carry):
            wx1, wy1, wx2, wy2, srv = carry
            warea = (wx2 - wx1) * (wy2 - wy1)

            @plsc.parallel_loop(0, nv, unroll=unroll,
                                carry=(jnp.full((16,), -jnp.inf),
                                       jnp.zeros((16,), jnp.int32)))
            def _scan(j, st):
                best, bidx = st
                o = j * 16
                x1 = cx1_v[pl.ds(o, 16)]
                y1 = cy1_v[pl.ds(o, 16)]
                x2 = cx2_v[pl.ds(o, 16)]
                y2 = cy2_v[pl.ds(o, 16)]
                s = cs_v[pl.ds(o, 16)]
                inter = (jnp.maximum(jnp.minimum(wx2, x2) -
                                     jnp.maximum(wx1, x1), 0.0) *
                         jnp.maximum(jnp.minimum(wy2, y2) -
                                     jnp.maximum(wy1, y1), 0.0))
                union = warea + (x2 - x1) * (y2 - y1) - inter
                iou = inter / jnp.maximum(union, 1e-9)
                snew = jnp.where(iou > _IOU_THRESH, -jnp.inf, s)
                cs_v[pl.ds(o, 16)] = snew
                jv = jnp.full((16,), j)
                upd = jnp.logical_or(
                    snew > best,
                    jnp.logical_and(snew == best, jv < bidx))
                best = jnp.where(upd, snew, best)
                bidx = jnp.where(upd, jv, bidx)
                return best, bidx

            best, bidx = _scan
            m = jnp.max(best)
            gidx = jnp.min(jnp.where(best == m, bidx * 16 + lane, _BIG))
            gidx = jnp.clip(gidx, 0, nv * 16 - 1)
            gv = jnp.full((16,), gidx)
            nwx1 = plsc.load_gather(cx1_v, [gv])
            nwy1 = plsc.load_gather(cy1_v, [gv])
            nwx2 = plsc.load_gather(cx2_v, [gv])
            nwy2 = plsc.load_gather(cy2_v, [gv])
            valid = m > -jnp.inf
            vb = jnp.full((16,), jnp.where(valid, 1.0, 0.0))
            vals = jnp.where(lane == 0, nwx1, 0.0)
            vals = jnp.where(lane == 1, nwy1, vals)
            vals = jnp.where(lane == 2, nwx2, vals)
            vals = jnp.where(lane == 3, nwy2, vals)
            vals = jnp.where(lane == 4, jnp.full((16,), m), vals)
            vals = jnp.where(vb > 0.5, vals, 0.0)
            out_v[pl.ds(i * 16, 16)] = vals
            return (nwx1, nwy1, nwx2, nwy2, srv + valid.astype(jnp.int32))

        init = (jnp.full((16,), -4000.0), jnp.full((16,), -4000.0),
                jnp.full((16,), -4000.0), jnp.full((16,), -4000.0),
                jnp.int32(0))
        return lax.fori_loop(0, _MAX_DET, _iter, init)[4]

    @pl.when(tid == 0)
    def _():
        nraw = _NTILES * _SLAB
        pltpu.sync_copy(cx1_sh.at[pl.ds(0, nraw)], cx1_v.at[pl.ds(0, nraw)])
        pltpu.sync_copy(cy1_sh.at[pl.ds(0, nraw)], cy1_v.at[pl.ds(0, nraw)])
        pltpu.sync_copy(cx2_sh.at[pl.ds(0, nraw)], cx2_v.at[pl.ds(0, nraw)])
        pltpu.sync_copy(cy2_sh.at[pl.ds(0, nraw)], cy2_v.at[pl.ds(0, nraw)])
        pltpu.sync_copy(cs_sh.at[pl.ds(0, nraw)], cs_v.at[pl.ds(0, nraw)])
        pltpu.sync_copy(rec_sh, recall_v)
        kvec = plsc.load_gather(recall_v, [lane * 16]).astype(jnp.int32)
        overflow = jnp.max(kvec) > _SLAB

        def _sq(j, off):
            o = j * 16
            sw = cs_v[pl.ds(o, 16)]
            m = sw > tthr
            plsc.store_compressed(cs_v.at[pl.ds(off, 16)], sw, mask=m)
            plsc.store_compressed(cx1_v.at[pl.ds(off, 16)],
                                  cx1_v[pl.ds(o, 16)], mask=m)
            plsc.store_compressed(cy1_v.at[pl.ds(off, 16)],
                                  cy1_v[pl.ds(o, 16)], mask=m)
            plsc.store_compressed(cx2_v.at[pl.ds(off, 16)],
                                  cx2_v[pl.ds(o, 16)], mask=m)
            plsc.store_compressed(cy2_v.at[pl.ds(off, 16)],
                                  cy2_v[pl.ds(o, 16)], mask=m)
            return off + jnp.max(plsc.all_reduce_population_count(m))
        ktot = lax.fori_loop(0, nraw // 16, _sq, jnp.int32(0))

        o0 = pl.multiple_of((ktot // 16) * 16, 16)
        tailv = cs_v[pl.ds(o0, 16)]
        cs_v[pl.ds(o0, 16)] = jnp.where(lane < ktot - o0, tailv, -jnp.inf)

        def _fill(j, c):
            cs_v[pl.ds(pl.multiple_of(o0 + 16 + j * 16, 16), 16)] = (
                jnp.full((16,), -jnp.inf))
            return c
        lax.fori_loop(0, jnp.maximum(_NV0 - o0 // 16 - 1, 0), _fill, 0)

        ok = jnp.logical_and(jnp.logical_not(overflow), ktot <= _NV0 * 16)
        srv = lax.cond(ok, lambda: _nms_loop(_NV0, 12), lambda: jnp.int32(0))
        need_full = jnp.logical_or(
            jnp.logical_not(ok),
            jnp.logical_and(srv < _MAX_DET, bstar >= _FULLBIN))
        flag_v[pl.ds(0, 16)] = jnp.full((16,), jnp.where(need_full, 1.0, 0.0))
        pltpu.sync_copy(flag_v, flag_sh)

    plsc.subcore_barrier()
    pltpu.sync_copy(flag_sh, flag_v)
    fl = jnp.max(flag_v[pl.ds(0, 16)])

    @pl.when(fl > 0.5)
    def _():
        pltpu.sync_copy(x1_v, cx1_sh.at[pl.ds(base, _PERTILE)])
        pltpu.sync_copy(y1_v, cy1_sh.at[pl.ds(base, _PERTILE)])
        pltpu.sync_copy(x2_v, cx2_sh.at[pl.ds(base, _PERTILE)])
        pltpu.sync_copy(y2_v, cy2_sh.at[pl.ds(base, _PERTILE)])
        pltpu.sync_copy(s_v, cs_sh.at[pl.ds(base, _PERTILE)])
        plsc.subcore_barrier()

        @pl.when(tid == 0)
        def _():
            pltpu.sync_copy(cx1_sh, cx1_v)
            pltpu.sync_copy(cy1_sh, cy1_v)
            pltpu.sync_copy(cx2_sh, cx2_v)
            pltpu.sync_copy(cy2_sh, cy2_v)
            pltpu.sync_copy(cs_sh, cs_v)
            _nms_loop(_NPAD // 16, 1)

    @pl.when(tid == 0)
    def _():
        pltpu.sync_copy(out_v, out_h)


@jax.jit
def kernel(boxes, scores):
    x1 = jnp.zeros((_NPAD,), jnp.float32).at[:_N].set(boxes[:, 0])
    y1 = jnp.zeros((_NPAD,), jnp.float32).at[:_N].set(boxes[:, 1])
    x2 = jnp.zeros((_NPAD,), jnp.float32).at[:_N].set(boxes[:, 2])
    y2 = jnp.zeros((_NPAD,), jnp.float32).at[:_N].set(boxes[:, 3])
    s = jnp.zeros((_NPAD,), jnp.float32).at[:_N].set(scores)
    out = _sc_nms(x1, y1, x2, y2, s)
    return out.reshape(_MAX_DET, 16)[:, :5]


# speedup vs baseline: 1.0458x; 1.0319x over previous
import functools

import jax
import jax.numpy as jnp
from jax import lax
from jax.experimental import pallas as pl
from jax.experimental.pallas import tpu as pltpu
from jax.experimental.pallas import tpu_sc as plsc

_N = 20000
_MAX_DET = 100
_IOU_THRESH = 0.5
_SCORE_THRESH = 0.05

_NTILES = 16
_PERTILE = 1280
_NPAD = _NTILES * _PERTILE
_VPT = _PERTILE // 16
_NBINS = 128
_C0 = 192
_SLAB = 32
_NV0 = 24
_FULLBIN = int(_SCORE_THRESH * _NBINS) + 1
_STG = _PERTILE + 16
_BIG = 2**30

_mesh = plsc.VectorSubcoreMesh(core_axis_name="c", subcore_axis_name="s",
                               num_cores=1)


@functools.partial(
    pl.kernel,
    out_type=jax.ShapeDtypeStruct((_MAX_DET * 16,), jnp.float32),
    mesh=_mesh,
    compiler_params=pltpu.CompilerParams(needs_layout_passes=False),
    scratch_types=dict(
        x1_v=pltpu.VMEM((_PERTILE,), jnp.float32),
        y1_v=pltpu.VMEM((_PERTILE,), jnp.float32),
        x2_v=pltpu.VMEM((_PERTILE,), jnp.float32),
        y2_v=pltpu.VMEM((_PERTILE,), jnp.float32),
        s_v=pltpu.VMEM((_PERTILE,), jnp.float32),
        hist_v=pltpu.VMEM((_NBINS,), jnp.float32),
        histall_v=pltpu.VMEM((_NTILES * _NBINS,), jnp.float32),
        recall_v=pltpu.VMEM((_NTILES * 16,), jnp.float32),
        stg_x1=pltpu.VMEM((_STG,), jnp.float32),
        stg_y1=pltpu.VMEM((_STG,), jnp.float32),
        stg_x2=pltpu.VMEM((_STG,), jnp.float32),
        stg_y2=pltpu.VMEM((_STG,), jnp.float32),
        stg_s=pltpu.VMEM((_STG,), jnp.float32),
        cx1_v=pltpu.VMEM((_NPAD,), jnp.float32),
        cy1_v=pltpu.VMEM((_NPAD,), jnp.float32),
        cx2_v=pltpu.VMEM((_NPAD,), jnp.float32),
        cy2_v=pltpu.VMEM((_NPAD,), jnp.float32),
        cs_v=pltpu.VMEM((_NPAD,), jnp.float32),
        ca_v=pltpu.VMEM((544,), jnp.float32),
        out_v=pltpu.VMEM((_MAX_DET * 16,), jnp.float32),
        rec_v=pltpu.VMEM((16,), jnp.float32),
        flag_v=pltpu.VMEM((16,), jnp.float32),
        hist_sh=pltpu.VMEM_SHARED((_NTILES * _NBINS,), jnp.float32),
        rec_sh=pltpu.VMEM_SHARED((_NTILES * 16,), jnp.float32),
        flag_sh=pltpu.VMEM_SHARED((16,), jnp.float32),
        cx1_sh=pltpu.VMEM_SHARED((_NPAD,), jnp.float32),
        cy1_sh=pltpu.VMEM_SHARED((_NPAD,), jnp.float32),
        cx2_sh=pltpu.VMEM_SHARED((_NPAD,), jnp.float32),
        cy2_sh=pltpu.VMEM_SHARED((_NPAD,), jnp.float32),
        cs_sh=pltpu.VMEM_SHARED((_NPAD,), jnp.float32),
    ),
)
def _sc_nms(x1_h, y1_h, x2_h, y2_h, s_h, out_h, *,
            x1_v, y1_v, x2_v, y2_v, s_v, hist_v, histall_v, recall_v,
            stg_x1, stg_y1, stg_x2, stg_y2, stg_s,
            cx1_v, cy1_v, cx2_v, cy2_v, cs_v, ca_v, out_v, rec_v, flag_v,
            hist_sh, rec_sh, flag_sh,
            cx1_sh, cy1_sh, cx2_sh, cy2_sh, cs_sh):
    tid = lax.axis_index("s")
    lane = lax.iota(jnp.int32, 16)
    lanef = lane.astype(jnp.float32)
    base = tid * _PERTILE

    pltpu.sync_copy(x1_h.at[pl.ds(base, _PERTILE)], x1_v)
    pltpu.sync_copy(y1_h.at[pl.ds(base, _PERTILE)], y1_v)
    pltpu.sync_copy(x2_h.at[pl.ds(base, _PERTILE)], x2_v)
    pltpu.sync_copy(y2_h.at[pl.ds(base, _PERTILE)], y2_v)
    pltpu.sync_copy(s_h.at[pl.ds(base, _PERTILE)], s_v)

    zeros16 = jnp.zeros((16,), jnp.float32)
    ones16 = jnp.ones((16,), jnp.float32)

    def _zero(k, c):
        hist_v[pl.ds(k * 16, 16)] = zeros16
        return c
    lax.fori_loop(0, _NBINS // 16, _zero, 0)

    @plsc.parallel_loop(0, _VPT, unroll=4, carry=jnp.int32(0))
    def _hstep(j, c):
        o = j * 16
        s = s_v[pl.ds(o, 16)]
        sw = jnp.where(s > _SCORE_THRESH, s, -jnp.inf)
        s_v[pl.ds(o, 16)] = sw
        m = sw > _SCORE_THRESH
        b = jnp.clip((sw * float(_NBINS)).astype(jnp.int32), 0, _NBINS - 1)
        plsc.addupdate_scatter(hist_v, [b], ones16, mask=m)
        return c
    _ = _hstep

    pltpu.sync_copy(hist_v, hist_sh.at[pl.ds(tid * _NBINS, _NBINS)])
    plsc.subcore_barrier()
    pltpu.sync_copy(hist_sh, histall_v)

    def _rsum(r, accs):
        return tuple(accs[k] + histall_v[pl.ds(r * _NBINS + k * 16, 16)]
                     for k in range(_NBINS // 16))
    hs = lax.fori_loop(0, _NTILES, _rsum,
                       tuple(zeros16 for _ in range(_NBINS // 16)))

    total = jnp.float32(0.0)
    bstar = jnp.int32(-1)
    for k in range(_NBINS // 16 - 1, -1, -1):
        v = hs[k]
        suf = lax.rev(plsc.cumsum(lax.rev(v, (0,))), (0,)) + total
        mask = suf >= float(_C0)
        bins = jnp.int32(k * 16) + lane
        bstar = jnp.maximum(bstar, jnp.max(jnp.where(mask, bins, -1)))
        total = total + jnp.sum(v)
    tthr = bstar.astype(jnp.float32) * (1.0 / _NBINS)

    for q in range(_SLAB // 16 + 1):
        stg_s[pl.ds(q * 16, 16)] = jnp.full((16,), -jnp.inf)

    def _cstep(j, off):
        o = j * 16
        sw = s_v[pl.ds(o, 16)]
        m = sw > tthr
        plsc.store_compressed(stg_s.at[pl.ds(off, 16)], sw, mask=m)
        plsc.store_compressed(stg_x1.at[pl.ds(off, 16)],
                              x1_v[pl.ds(o, 16)], mask=m)
        plsc.store_compressed(stg_y1.at[pl.ds(off, 16)],
                              y1_v[pl.ds(o, 16)], mask=m)
        plsc.store_compressed(stg_x2.at[pl.ds(off, 16)],
                              x2_v[pl.ds(o, 16)], mask=m)
        plsc.store_compressed(stg_y2.at[pl.ds(off, 16)],
                              y2_v[pl.ds(o, 16)], mask=m)
        return off + jnp.max(plsc.all_reduce_population_count(m))
    kcnt = lax.fori_loop(0, _VPT, _cstep, jnp.int32(0))

    pltpu.sync_copy(stg_x1.at[pl.ds(0, _SLAB)],
                    cx1_sh.at[pl.ds(tid * _SLAB, _SLAB)])
    pltpu.sync_copy(stg_y1.at[pl.ds(0, _SLAB)],
                    cy1_sh.at[pl.ds(tid * _SLAB, _SLAB)])
    pltpu.sync_copy(stg_x2.at[pl.ds(0, _SLAB)],
                    cx2_sh.at[pl.ds(tid * _SLAB, _SLAB)])
    pltpu.sync_copy(stg_y2.at[pl.ds(0, _SLAB)],
                    cy2_sh.at[pl.ds(tid * _SLAB, _SLAB)])
    pltpu.sync_copy(stg_s.at[pl.ds(0, _SLAB)],
                    cs_sh.at[pl.ds(tid * _SLAB, _SLAB)])
    rec_v[pl.ds(0, 16)] = jnp.where(lane == 0, kcnt.astype(jnp.float32), 0.0)
    pltpu.sync_copy(rec_v, rec_sh.at[pl.ds(tid * 16, 16)])
    plsc.subcore_barrier()

    def _nms_loop(nv, unroll, use_area):
        def _iter(i, carry):
            wx1, wy1, wx2, wy2, warea, srv = carry

            @plsc.parallel_loop(0, nv, unroll=unroll,
                                carry=(jnp.full((16,), -jnp.inf),
                                       jnp.zeros((16,), jnp.int32)))
            def _scan(j, st):
                best, bidx = st
                o = j * 16
                x1 = cx1_v[pl.ds(o, 16)]
                y1 = cy1_v[pl.ds(o, 16)]
                x2 = cx2_v[pl.ds(o, 16)]
                y2 = cy2_v[pl.ds(o, 16)]
                s = cs_v[pl.ds(o, 16)]
                a = (ca_v[pl.ds(o, 16)] if use_area
                     else (x2 - x1) * (y2 - y1))
                inter = (jnp.maximum(jnp.minimum(wx2, x2) -
                                     jnp.maximum(wx1, x1), 0.0) *
                         jnp.maximum(jnp.minimum(wy2, y2) -
                                     jnp.maximum(wy1, y1), 0.0))
                union = warea + a - inter
                iou = inter / jnp.maximum(union, 1e-9)
                snew = jnp.where(iou > _IOU_THRESH, -jnp.inf, s)
                cs_v[pl.ds(o, 16)] = snew
                jv = jnp.full((16,), j)
                upd = jnp.logical_or(
                    snew > best,
                    jnp.logical_and(snew == best, jv < bidx))
                best = jnp.where(upd, snew, best)
                bidx = jnp.where(upd, jv, bidx)
                return best, bidx

            best, bidx = _scan
            m = jnp.max(best)
            gidx = jnp.min(jnp.where(best == m, bidx * 16 + lane, _BIG))
            gidx = jnp.clip(gidx, 0, nv * 16 - 1)
            gv = jnp.full((16,), gidx)
            nwx1 = plsc.load_gather(cx1_v, [gv])
            nwy1 = plsc.load_gather(cy1_v, [gv])
            nwx2 = plsc.load_gather(cx2_v, [gv])
            nwy2 = plsc.load_gather(cy2_v, [gv])
            nwa = (plsc.load_gather(ca_v, [gv]) if use_area
                   else (nwx2 - nwx1) * (nwy2 - nwy1))
            valid = m > -jnp.inf
            vb = jnp.full((16,), jnp.where(valid, 1.0, 0.0))
            vals = jnp.where(lane == 0, nwx1, 0.0)
            vals = jnp.where(lane == 1, nwy1, vals)
            vals = jnp.where(lane == 2, nwx2, vals)
            vals = jnp.where(lane == 3, nwy2, vals)
            vals = jnp.where(lane == 4, jnp.full((16,), m), vals)
            vals = jnp.where(vb > 0.5, vals, 0.0)
            out_v[pl.ds(i * 16, 16)] = vals
            return (nwx1, nwy1, nwx2, nwy2, nwa,
                    srv + valid.astype(jnp.int32))

        init = (jnp.full((16,), -4000.0), jnp.full((16,), -4000.0),
                jnp.full((16,), -4000.0), jnp.full((16,), -4000.0),
                jnp.zeros((16,)), jnp.int32(0))
        return lax.fori_loop(0, _MAX_DET, _iter, init)[5]

    @pl.when(tid == 0)
    def _():
        nraw = _NTILES * _SLAB
        pltpu.sync_copy(cx1_sh.at[pl.ds(0, nraw)], cx1_v.at[pl.ds(0, nraw)])
        pltpu.sync_copy(cy1_sh.at[pl.ds(0, nraw)], cy1_v.at[pl.ds(0, nraw)])
        pltpu.sync_copy(cx2_sh.at[pl.ds(0, nraw)], cx2_v.at[pl.ds(0, nraw)])
        pltpu.sync_copy(cy2_sh.at[pl.ds(0, nraw)], cy2_v.at[pl.ds(0, nraw)])
        pltpu.sync_copy(cs_sh.at[pl.ds(0, nraw)], cs_v.at[pl.ds(0, nraw)])
        pltpu.sync_copy(rec_sh, recall_v)
        kvec = plsc.load_gather(recall_v, [lane * 16]).astype(jnp.int32)
        overflow = jnp.max(kvec) > _SLAB

        def _sq(j, off):
            o = j * 16
            sw = cs_v[pl.ds(o, 16)]
            m = sw > tthr
            x1 = cx1_v[pl.ds(o, 16)]
            y1 = cy1_v[pl.ds(o, 16)]
            x2 = cx2_v[pl.ds(o, 16)]
            y2 = cy2_v[pl.ds(o, 16)]
            plsc.store_compressed(cs_v.at[pl.ds(off, 16)], sw, mask=m)
            plsc.store_compressed(cx1_v.at[pl.ds(off, 16)], x1, mask=m)
            plsc.store_compressed(cy1_v.at[pl.ds(off, 16)], y1, mask=m)
            plsc.store_compressed(cx2_v.at[pl.ds(off, 16)], x2, mask=m)
            plsc.store_compressed(cy2_v.at[pl.ds(off, 16)], y2, mask=m)
            plsc.store_compressed(ca_v.at[pl.ds(off, 16)],
                                  (x2 - x1) * (y2 - y1), mask=m)
            return off + jnp.max(plsc.all_reduce_population_count(m))
        ktot = lax.fori_loop(0, nraw // 16, _sq, jnp.int32(0))

        o0 = pl.multiple_of((ktot // 16) * 16, 16)
        tailv = cs_v[pl.ds(o0, 16)]
        cs_v[pl.ds(o0, 16)] = jnp.where(lane < ktot - o0, tailv, -jnp.inf)

        def _fill(j, c):
            cs_v[pl.ds(pl.multiple_of(o0 + 16 + j * 16, 16), 16)] = (
                jnp.full((16,), -jnp.inf))
            return c
        lax.fori_loop(0, jnp.maximum(_NV0 - o0 // 16 - 1, 0), _fill, 0)

        ok = jnp.logical_and(jnp.logical_not(overflow), ktot <= _NV0 * 16)
        srv = lax.cond(ok, lambda: _nms_loop(_NV0, 12, True),
                       lambda: jnp.int32(0))
        need_full = jnp.logical_or(
            jnp.logical_not(ok),
            jnp.logical_and(srv < _MAX_DET, bstar >= _FULLBIN))
        flag_v[pl.ds(0, 16)] = jnp.full((16,), jnp.where(need_full, 1.0, 0.0))
        pltpu.sync_copy(flag_v, flag_sh)

    plsc.subcore_barrier()
    pltpu.sync_copy(flag_sh, flag_v)
    fl = jnp.max(flag_v[pl.ds(0, 16)])

    @pl.when(fl > 0.5)
    def _():
        pltpu.sync_copy(x1_v, cx1_sh.at[pl.ds(base, _PERTILE)])
        pltpu.sync_copy(y1_v, cy1_sh.at[pl.ds(base, _PERTILE)])
        pltpu.sync_copy(x2_v, cx2_sh.at[pl.ds(base, _PERTILE)])
        pltpu.sync_copy(y2_v, cy2_sh.at[pl.ds(base, _PERTILE)])
        pltpu.sync_copy(s_v, cs_sh.at[pl.ds(base, _PERTILE)])
        plsc.subcore_barrier()

        @pl.when(tid == 0)
        def _():
            pltpu.sync_copy(cx1_sh, cx1_v)
            pltpu.sync_copy(cy1_sh, cy1_v)
            pltpu.sync_copy(cx2_sh, cx2_v)
            pltpu.sync_copy(cy2_sh, cy2_v)
            pltpu.sync_copy(cs_sh, cs_v)
            _nms_loop(_NPAD // 16, 1, False)

    @pl.when(tid == 0)
    def _():
        pltpu.sync_copy(out_v, out_h)


@jax.jit
def kernel(boxes, scores):
    x1 = jnp.zeros((_NPAD,), jnp.float32).at[:_N].set(boxes[:, 0])
    y1 = jnp.zeros((_NPAD,), jnp.float32).at[:_N].set(boxes[:, 1])
    x2 = jnp.zeros((_NPAD,), jnp.float32).at[:_N].set(boxes[:, 2])
    y2 = jnp.zeros((_NPAD,), jnp.float32).at[:_N].set(boxes[:, 3])
    s = jnp.zeros((_NPAD,), jnp.float32).at[:_N].set(scores)
    out = _sc_nms(x1, y1, x2, y2, s)
    return out.reshape(_MAX_DET, 16)[:, :5]
